# baseline (speedup 1.0000x reference)
import functools

import jax
import jax.numpy as jnp
from jax import lax
from jax.experimental import pallas as pl
from jax.experimental.pallas import tpu as pltpu
from jax.experimental.pallas import tpu_sc as plsc

PC_START = (-54.0, -54.0)
VOXEL_SIZE = (0.075, 0.075)
OUT_STRIDE = 8

B = 4
H = 180
W = 180
C = 512
SUB = C // 128
N = 2500
NSEC = 5
SEC = N // NSEC
NPTS = B * N

NC = 2
NS = 16
LANES = 16
PPW = 320
PPH = 16
NH = PPW // PPH
XPAD = PPW + PPH


def _sc_interp(bev2, xs, ys, rowbase):
    mesh = plsc.VectorSubcoreMesh(core_axis_name="c", subcore_axis_name="s")

    @functools.partial(
        pl.kernel,
        mesh=mesh,
        out_type=jax.ShapeDtypeStruct((NPTS * SUB, 128), jnp.float32),
        scratch_types=[
            pltpu.VMEM((XPAD,), jnp.float32),
            pltpu.VMEM((XPAD,), jnp.float32),
            pltpu.VMEM((XPAD,), jnp.int32),
            pltpu.VMEM((2 * PPH * SUB,), jnp.int32),
            pltpu.VMEM((2 * PPH * SUB,), jnp.int32),
            pltpu.VMEM((2 * PPH * SUB,), jnp.int32),
            pltpu.VMEM((2 * PPH * SUB,), jnp.int32),
            pltpu.VMEM((4 * PPH, LANES), jnp.float32),
            pltpu.VMEM((4 * PPH, LANES), jnp.float32),
            pltpu.VMEM((4 * PPH * SUB, 128), jnp.float32),
            pltpu.VMEM((4 * PPH * SUB, 128), jnp.float32),
            pltpu.VMEM((PPH * SUB, 128), jnp.float32),
            pltpu.VMEM((PPH * SUB, 128), jnp.float32),
            pltpu.SemaphoreType.DMA,
            pltpu.SemaphoreType.DMA,
            pltpu.SemaphoreType.DMA,
            pltpu.SemaphoreType.DMA,
        ],
    )
    def k(bev_hbm, xs_hbm, ys_hbm, base_hbm, out_hbm,
          xv, yv, bv, ixa0, ixa1, ixb0, ixb1, wts_a, wts_b, rows_a, rows_b,
          out_a, out_b, gs_a, gs_b, os_a, os_b):
        wid = lax.axis_index("s") * NC + lax.axis_index("c")
        wbase = jnp.minimum(wid * PPW, NPTS - PPW)
        pltpu.sync_copy(xs_hbm.at[pl.ds(wbase, PPW)], xv.at[pl.ds(0, PPW)])
        pltpu.sync_copy(ys_hbm.at[pl.ds(wbase, PPW)], yv.at[pl.ds(0, PPW)])
        pltpu.sync_copy(base_hbm.at[pl.ds(wbase, PPW)], bv.at[pl.ds(0, PPW)])
        xv[pl.ds(PPW, PPH)] = xv[pl.ds(0, PPH)]
        yv[pl.ds(PPW, PPH)] = yv[pl.ds(0, PPH)]
        bv[pl.ds(PPW, PPH)] = bv[pl.ds(0, PPH)]

        def compute_idx(h, ix0, ix1, wts_v):
            o = h * PPH
            x = xv[pl.ds(o, LANES)]
            y = yv[pl.ds(o, LANES)]
            bb = bv[pl.ds(o, LANES)]
            x = (x - PC_START[0]) / VOXEL_SIZE[0] / OUT_STRIDE
            y = (y - PC_START[1]) / VOXEL_SIZE[1] / OUT_STRIDE
            x = jnp.minimum(jnp.maximum(x, -4.0), 184.0)
            y = jnp.minimum(jnp.maximum(y, -4.0), 184.0)
            xi = x.astype(jnp.int32)
            yi = y.astype(jnp.int32)
            x0 = jnp.where(xi.astype(jnp.float32) > x, xi - 1, xi)
            y0 = jnp.where(yi.astype(jnp.float32) > y, yi - 1, yi)
            x0c = jnp.minimum(jnp.maximum(x0, 0), W - 1)
            x1c = jnp.minimum(jnp.maximum(x0 + 1, 0), W - 1)
            y0c = jnp.minimum(jnp.maximum(y0, 0), H - 1)
            y1c = jnp.minimum(jnp.maximum(y0 + 1, 0), H - 1)
            fx0 = x0c.astype(jnp.float32)
            fx1 = x1c.astype(jnp.float32)
            fy0 = y0c.astype(jnp.float32)
            fy1 = y1c.astype(jnp.float32)
            wa = (fx1 - x) * (fy1 - y)
            wb = (fx1 - x) * (y - fy0)
            wc = (x - fx0) * (fy1 - y)
            wd = (x - fx0) * (y - fy0)
            for l in range(LANES):
                wts_v[0 * PPH + l, :] = jnp.full((LANES,), wa[l])
                wts_v[1 * PPH + l, :] = jnp.full((LANES,), wb[l])
                wts_v[2 * PPH + l, :] = jnp.full((LANES,), wc[l])
                wts_v[3 * PPH + l, :] = jnp.full((LANES,), wd[l])
            row0 = bb + y0c * W
            row1 = bb + y1c * W
            ra = (row0 + x0c) * SUB
            rb = (row1 + x0c) * SUB
            rc = (row0 + x1c) * SUB
            rd = (row1 + x1c) * SUB
            for lc in range(SUB):
                ix0[pl.ds((0 * SUB + lc) * LANES, LANES)] = ra + lc
                ix0[pl.ds((1 * SUB + lc) * LANES, LANES)] = rb + lc
                ix1[pl.ds((0 * SUB + lc) * LANES, LANES)] = rc + lc
                ix1[pl.ds((1 * SUB + lc) * LANES, LANES)] = rd + lc

        HALF_ROWS = 2 * PPH * SUB

        def fire_gather(ix0, ix1, rows_v, sem):
            pltpu.async_copy(bev_hbm.at[ix0], rows_v.at[pl.ds(0, HALF_ROWS)], sem)
            pltpu.async_copy(bev_hbm.at[ix1], rows_v.at[pl.ds(HALF_ROWS, HALF_ROWS)], sem)

        def wait_gather(ix0, ix1, rows_v, sem):
            pltpu.make_async_copy(bev_hbm.at[ix0], rows_v.at[pl.ds(0, HALF_ROWS)], sem).wait()
            pltpu.make_async_copy(bev_hbm.at[ix1], rows_v.at[pl.ds(HALF_ROWS, HALF_ROWS)], sem).wait()

        def blend(rows_v, wts_v, out_v):
            for p0 in range(0, PPH, 8):
                wregs = [[wts_v[c * PPH + p0 + p, :] for p in range(8)]
                         for c in range(4)]
                for lc in range(SUB):
                    @plsc.parallel_loop(0, 128 // LANES, unroll=2)
                    def body(jj, _wregs=wregs, _p0=p0, _lc=lc):
                        s = pl.ds(jj * LANES, LANES)
                        for p in range(8):
                            kk = _p0 + p
                            va = rows_v[(0 * SUB + _lc) * PPH + kk, s]
                            vb = rows_v[(1 * SUB + _lc) * PPH + kk, s]
                            vc = rows_v[(2 * SUB + _lc) * PPH + kk, s]
                            vd = rows_v[(3 * SUB + _lc) * PPH + kk, s]
                            out_v[kk * SUB + _lc, s] = (
                                ((va * _wregs[0][p] + vb * _wregs[1][p])
                                 + vc * _wregs[2][p]) + vd * _wregs[3][p])

        compute_idx(0, ixa0, ixa1, wts_a)
        fire_gather(ixa0, ixa1, rows_a, gs_a)

        def pair_body(i, _):
            compute_idx(2 * i + 1, ixb0, ixb1, wts_b)
            fire_gather(ixb0, ixb1, rows_b, gs_b)

            wait_gather(ixa0, ixa1, rows_a, gs_a)

            @pl.when(i != 0)
            def _():
                pltpu.make_async_copy(out_a, out_hbm.at[pl.ds(wbase * SUB, PPH * SUB)], os_a).wait()
            blend(rows_a, wts_a, out_a)
            pltpu.async_copy(
                out_a, out_hbm.at[pl.ds((wbase + (2 * i) * PPH) * SUB, PPH * SUB)], os_a)

            compute_idx(2 * i + 2, ixa0, ixa1, wts_a)
            fire_gather(ixa0, ixa1, rows_a, gs_a)

            wait_gather(ixb0, ixb1, rows_b, gs_b)

            @pl.when(i != 0)
            def _():
                pltpu.make_async_copy(out_b, out_hbm.at[pl.ds(wbase * SUB, PPH * SUB)], os_b).wait()
            blend(rows_b, wts_b, out_b)
            pltpu.async_copy(
                out_b, out_hbm.at[pl.ds((wbase + (2 * i + 1) * PPH) * SUB, PPH * SUB)], os_b)
            return 0

        lax.fori_loop(0, NH // 2, pair_body, 0)

        wait_gather(ixa0, ixa1, rows_a, gs_a)
        pltpu.make_async_copy(out_a, out_hbm.at[pl.ds(wbase * SUB, PPH * SUB)], os_a).wait()
        pltpu.make_async_copy(out_b, out_hbm.at[pl.ds(wbase * SUB, PPH * SUB)], os_b).wait()

    return k(bev2, xs, ys, rowbase)


def kernel(bev_feature, batch_centers, num_point):
    del num_point
    cx = batch_centers[..., 0]
    cy = batch_centers[..., 1]
    order = (jnp.arange(SEC)[:, None] + SEC * jnp.arange(NSEC)[None, :]).reshape(-1)
    xs = cx[:, order].reshape(-1)
    ys = cy[:, order].reshape(-1)
    rowbase = jnp.repeat(jnp.arange(B, dtype=jnp.int32) * (H * W), N)
    bev2 = bev_feature.reshape(B * H * W * SUB, 128)
    out = _sc_interp(bev2, xs, ys, rowbase)
    return out.reshape(B, SEC, NSEC * C)
